# baseline (speedup 1.0000x reference)
import functools

import jax
import jax.numpy as jnp
from jax import lax
from jax.experimental import pallas as pl
from jax.experimental.pallas import tpu as pltpu
from jax.experimental.pallas import tpu_sc as plsc

NUM_CORES = 2
NUM_SUBCORES = 16
NUM_WORKERS = NUM_CORES * NUM_SUBCORES
TBLK = 512
NBUF = 5
CHUNK = 128


def _transpose_table(emb_t):
    feats, v = emb_t.shape
    n_blk = pl.cdiv(v, TBLK)
    vpad = n_blk * TBLK

    def body(in_ref, out_ref):
        t = in_ref[...].T
        out_ref[...] = jnp.concatenate(
            [t, jnp.zeros((TBLK, 128 - feats), jnp.float32)], axis=1
        )

    return pl.pallas_call(
        body,
        grid=(n_blk,),
        in_specs=[pl.BlockSpec((feats, TBLK), lambda i: (0, i))],
        out_specs=pl.BlockSpec((TBLK, 128), lambda i: (i, 0)),
        out_shape=jax.ShapeDtypeStruct((vpad, 128), jnp.float32),
    )(emb_t)


def _transpose_out(rows, b0, b1, features):
    n_chunks = rows.shape[0]
    chunks_per_blk = b1
    n_blk = n_chunks // chunks_per_blk

    def body(in_ref, out_ref):
        v = in_ref[:, :, :features].reshape(CHUNK, b1, features)
        out_ref[...] = jnp.transpose(v, (1, 2, 0))

    return pl.pallas_call(
        body,
        grid=(n_blk,),
        in_specs=[
            pl.BlockSpec((chunks_per_blk, CHUNK, 128), lambda i: (i, 0, 0))
        ],
        out_specs=pl.BlockSpec((b1, features, CHUNK), lambda i: (0, 0, i)),
        out_shape=jax.ShapeDtypeStruct((b1, features, b0), jnp.float32),
    )(rows)


def kernel(x, embedding):
    b0, b1 = x.shape
    features = embedding.shape[1]
    total = b0 * b1
    n_chunks = total // CHUNK
    per_w = n_chunks // NUM_WORKERS
    n_groups = per_w // NBUF

    idx = x.reshape(n_chunks, CHUNK).astype(jnp.int32)
    table = _transpose_table(embedding.T)

    mesh = plsc.VectorSubcoreMesh(core_axis_name="c", subcore_axis_name="s")

    @functools.partial(
        pl.kernel,
        out_type=jax.ShapeDtypeStruct((n_chunks, CHUNK, 128), jnp.float32),
        mesh=mesh,
        scratch_types=[
            pltpu.VMEM((per_w, CHUNK), jnp.int32),
            pltpu.VMEM((NBUF, CHUNK, 128), jnp.float32),
            pltpu.SemaphoreType.DMA((NBUF,)),
            pltpu.SemaphoreType.DMA((NBUF,)),
        ],
        compiler_params=pltpu.CompilerParams(needs_layout_passes=False),
    )
    def embed_gather(table_hbm, idx_hbm, out_hbm, idx_v, rows_v, gsem, wsem):
        w = lax.axis_index("s") * NUM_CORES + lax.axis_index("c")
        base = w * per_w
        pltpu.sync_copy(idx_hbm.at[pl.ds(base, per_w)], idx_v)

        def group(g0, carry):
            k0 = g0 * NBUF
            gathers = []
            for b in range(NBUF):
                gathers.append(
                    pltpu.async_copy(
                        table_hbm.at[idx_v.at[k0 + b]],
                        rows_v.at[b],
                        gsem.at[b],
                    )
                )
            writes = []
            for b in range(NBUF):
                gathers[b].wait()
                writes.append(
                    pltpu.async_copy(
                        rows_v.at[b],
                        out_hbm.at[base + k0 + b],
                        wsem.at[b],
                    )
                )
            for wd in writes:
                wd.wait()
            return carry

        lax.fori_loop(0, n_groups, group, 0)

    rows = embed_gather(table, idx)
    out_t = _transpose_out(rows, b0, b1, features)
    return jnp.transpose(out_t, (2, 0, 1))


# speedup vs baseline: 1.3467x; 1.3467x over previous
import functools

import jax
import jax.numpy as jnp
from jax import lax
from jax.experimental import pallas as pl
from jax.experimental.pallas import tpu as pltpu
from jax.experimental.pallas import tpu_sc as plsc

NUM_CORES = 2
NUM_SUBCORES = 16
NUM_WORKERS = NUM_CORES * NUM_SUBCORES
TBLK = 1024
NBUF = 8
CHUNK = 128


def _transpose_table(emb_t):
    feats, v = emb_t.shape
    half = pl.cdiv(v, 2 * TBLK) * TBLK
    n_blk = half // TBLK
    vpad = 2 * half
    last_blk = pl.cdiv(v, TBLK) - 1

    def body(lo_ref, hi_ref, out_ref):
        out_ref[...] = jnp.concatenate([lo_ref[...].T, hi_ref[...].T], axis=1)

    packed = pl.pallas_call(
        body,
        grid=(n_blk,),
        in_specs=[
            pl.BlockSpec((feats, TBLK), lambda i: (0, i)),
            pl.BlockSpec(
                (feats, TBLK),
                lambda i, nb=n_blk, lb=last_blk: (0, jnp.minimum(nb + i, lb)),
            ),
        ],
        out_specs=pl.BlockSpec((TBLK, 2 * feats), lambda i: (i, 0)),
        out_shape=jax.ShapeDtypeStruct((half, 2 * feats), jnp.float32),
    )(emb_t, emb_t)
    return packed.reshape(vpad, feats), half


def _transpose_out(rows_packed, b0, b1, features):
    rows_per_blk = b1 * features
    n_blk = rows_packed.shape[0] // rows_per_blk

    def body(in_ref, out_ref):
        v = in_ref[...].reshape(b1, features, 2 * features)
        ve = v[:, :, :features].reshape(b1, features, 1, features)
        vo = v[:, :, features:].reshape(b1, features, 1, features)
        flat = jnp.concatenate([ve, vo], axis=2).reshape(CHUNK, b1, features)
        out_ref[...] = jnp.transpose(flat, (1, 2, 0))

    return pl.pallas_call(
        body,
        grid=(n_blk,),
        in_specs=[pl.BlockSpec((rows_per_blk, 128), lambda i: (i, 0))],
        out_specs=pl.BlockSpec((b1, features, CHUNK), lambda i: (0, 0, i)),
        out_shape=jax.ShapeDtypeStruct((b1, features, b0), jnp.float32),
    )(rows_packed)


def kernel(x, embedding):
    b0, b1 = x.shape
    features = embedding.shape[1]
    total = b0 * b1
    n_chunks = total // CHUNK
    per_w = n_chunks // NUM_WORKERS
    n_groups = per_w // NBUF

    table, half = _transpose_table(embedding.T)
    xi = x.astype(jnp.int32)
    idx = jnp.where(xi < half, 2 * xi, 2 * (xi - half) + 1).reshape(
        n_chunks, CHUNK
    )

    mesh = plsc.VectorSubcoreMesh(core_axis_name="c", subcore_axis_name="s")

    @functools.partial(
        pl.kernel,
        out_type=jax.ShapeDtypeStruct((n_chunks, CHUNK, features), jnp.float32),
        mesh=mesh,
        scratch_types=[
            pltpu.VMEM((per_w, CHUNK), jnp.int32),
            pltpu.VMEM((NBUF, CHUNK, features), jnp.float32),
            pltpu.SemaphoreType.DMA((NBUF,)),
            pltpu.SemaphoreType.DMA((NBUF,)),
        ],
        compiler_params=pltpu.CompilerParams(use_tc_tiling_on_sc=False),
    )
    def embed_gather(table_hbm, idx_hbm, out_hbm, idx_v, rows_v, gsem, wsem):
        w = lax.axis_index("s") * NUM_CORES + lax.axis_index("c")
        base = w * per_w
        pltpu.sync_copy(idx_hbm.at[pl.ds(base, per_w)], idx_v)

        def group(g0, carry):
            k0 = g0 * NBUF
            gathers = []
            for b in range(NBUF):
                gathers.append(
                    pltpu.async_copy(
                        table_hbm.at[idx_v.at[k0 + b]],
                        rows_v.at[b],
                        gsem.at[b],
                    )
                )
            writes = []
            for b in range(NBUF):
                gathers[b].wait()
                writes.append(
                    pltpu.async_copy(
                        rows_v.at[b],
                        out_hbm.at[base + k0 + b],
                        wsem.at[b],
                    )
                )
            for wd in writes:
                wd.wait()
            return carry

        lax.fori_loop(0, n_groups, group, 0)

    rows = embed_gather(table, idx)
    rows_packed = rows.reshape(total * features // 128, 128)
    out_t = _transpose_out(rows_packed, b0, b1, features)
    return jnp.transpose(out_t, (2, 0, 1))


# speedup vs baseline: 1.5736x; 1.1685x over previous
import functools

import jax
import jax.numpy as jnp
from jax import lax
from jax.experimental import pallas as pl
from jax.experimental.pallas import tpu as pltpu
from jax.experimental.pallas import tpu_sc as plsc

NUM_CORES = 2
NUM_SUBCORES = 16
NUM_WORKERS = NUM_CORES * NUM_SUBCORES
TBLK = 4096
NBUF = 8
CHUNK = 128


def _transpose_table(emb_t):
    feats, v = emb_t.shape
    half = pl.cdiv(v, 2 * TBLK) * TBLK
    n_blk = half // TBLK
    vpad = 2 * half
    last_blk = pl.cdiv(v, TBLK) - 1

    def body(lo_ref, hi_ref, out_ref):
        out_ref[...] = jnp.concatenate([lo_ref[...].T, hi_ref[...].T], axis=1)

    packed = pl.pallas_call(
        body,
        grid=(n_blk,),
        in_specs=[
            pl.BlockSpec((feats, TBLK), lambda i: (0, i)),
            pl.BlockSpec(
                (feats, TBLK),
                lambda i, nb=n_blk, lb=last_blk: (0, jnp.minimum(nb + i, lb)),
            ),
        ],
        out_specs=pl.BlockSpec((TBLK, 2 * feats), lambda i: (i, 0)),
        out_shape=jax.ShapeDtypeStruct((half, 2 * feats), jnp.float32),
    )(emb_t, emb_t)
    return packed.reshape(vpad, feats), half


def _transpose_out(rows_packed, b0, b1, features):
    rows_per_blk = b1 * features
    n_blk = rows_packed.shape[0] // rows_per_blk

    group = 4

    def body(in_ref, out_ref):
        for s in range(group):
            v = in_ref[pl.ds(s * rows_per_blk, rows_per_blk), :].reshape(
                b1, features, 2 * features
            )
            ve = v[:, :, :features].reshape(b1, features, 1, features)
            vo = v[:, :, features:].reshape(b1, features, 1, features)
            flat = jnp.concatenate([ve, vo], axis=2).reshape(
                CHUNK, b1, features
            )
            out_ref[:, :, pl.ds(s * CHUNK, CHUNK)] = jnp.transpose(
                flat, (1, 2, 0)
            )

    return pl.pallas_call(
        body,
        grid=(n_blk // group,),
        in_specs=[pl.BlockSpec((group * rows_per_blk, 128), lambda i: (i, 0))],
        out_specs=pl.BlockSpec(
            (b1, features, group * CHUNK), lambda i: (0, 0, i)
        ),
        out_shape=jax.ShapeDtypeStruct((b1, features, b0), jnp.float32),
    )(rows_packed)


def kernel(x, embedding):
    b0, b1 = x.shape
    features = embedding.shape[1]
    total = b0 * b1
    n_chunks = total // CHUNK
    per_w = n_chunks // NUM_WORKERS
    n_groups = per_w // NBUF

    table, half = _transpose_table(embedding.T)
    xi = x.astype(jnp.int32)
    idx = jnp.where(xi < half, 2 * xi, 2 * (xi - half) + 1).reshape(
        n_chunks, CHUNK
    )

    mesh = plsc.VectorSubcoreMesh(core_axis_name="c", subcore_axis_name="s")

    @functools.partial(
        pl.kernel,
        out_type=jax.ShapeDtypeStruct((n_chunks, CHUNK, features), jnp.float32),
        mesh=mesh,
        scratch_types=[
            pltpu.VMEM((per_w, CHUNK), jnp.int32),
            pltpu.VMEM((NBUF, CHUNK, features), jnp.float32),
            pltpu.SemaphoreType.DMA((NBUF,)),
            pltpu.SemaphoreType.DMA((NBUF,)),
        ],
        compiler_params=pltpu.CompilerParams(use_tc_tiling_on_sc=False),
    )
    def embed_gather(table_hbm, idx_hbm, out_hbm, idx_v, rows_v, gsem, wsem):
        w = lax.axis_index("s") * NUM_CORES + lax.axis_index("c")
        base = w * per_w
        pltpu.sync_copy(idx_hbm.at[pl.ds(base, per_w)], idx_v)

        def group(g0, carry):
            k0 = g0 * NBUF
            gathers = []
            for b in range(NBUF):
                gathers.append(
                    pltpu.async_copy(
                        table_hbm.at[idx_v.at[k0 + b]],
                        rows_v.at[b],
                        gsem.at[b],
                    )
                )
            writes = []
            for b in range(NBUF):
                gathers[b].wait()
                writes.append(
                    pltpu.async_copy(
                        rows_v.at[b],
                        out_hbm.at[base + k0 + b],
                        wsem.at[b],
                    )
                )
            for wd in writes:
                wd.wait()
            return carry

        lax.fori_loop(0, n_groups, group, 0)

    rows = embed_gather(table, idx)
    rows_packed = rows.reshape(total * features // 128, 128)
    out_t = _transpose_out(rows_packed, b0, b1, features)
    return jnp.transpose(out_t, (2, 0, 1))


# speedup vs baseline: 1.6163x; 1.0271x over previous
import functools

import jax
import jax.numpy as jnp
from jax import lax
from jax.experimental import pallas as pl
from jax.experimental.pallas import tpu as pltpu
from jax.experimental.pallas import tpu_sc as plsc

NUM_CORES = 2
NUM_SUBCORES = 16
NUM_WORKERS = NUM_CORES * NUM_SUBCORES
TBLK = 8192
NBUF = 8
CHUNK = 128


def _transpose_table(emb_t):
    feats, v = emb_t.shape
    half = pl.cdiv(v, 2 * TBLK) * TBLK
    n_blk = half // TBLK
    vpad = 2 * half
    last_blk = pl.cdiv(v, TBLK) - 1

    def body(lo_ref, hi_ref, out_ref):
        out_ref[...] = jnp.concatenate([lo_ref[...].T, hi_ref[...].T], axis=1)

    packed = pl.pallas_call(
        body,
        grid=(n_blk,),
        in_specs=[
            pl.BlockSpec((feats, TBLK), lambda i: (0, i)),
            pl.BlockSpec(
                (feats, TBLK),
                lambda i, nb=n_blk, lb=last_blk: (0, jnp.minimum(nb + i, lb)),
            ),
        ],
        out_specs=pl.BlockSpec((TBLK, 2 * feats), lambda i: (i, 0)),
        out_shape=jax.ShapeDtypeStruct((half, 2 * feats), jnp.float32),
    )(emb_t, emb_t)
    return packed.reshape(vpad, feats), half


def _transpose_out(rows_packed, b0, b1, features):
    rows_per_blk = b1 * features
    n_blk = rows_packed.shape[0] // rows_per_blk

    group = 4

    def body(in_ref, out_ref):
        for s in range(group):
            v = in_ref[pl.ds(s * rows_per_blk, rows_per_blk), :].reshape(
                b1, features, 2 * features
            )
            ve = v[:, :, :features].reshape(b1, features, 1, features)
            vo = v[:, :, features:].reshape(b1, features, 1, features)
            flat = jnp.concatenate([ve, vo], axis=2).reshape(
                CHUNK, b1, features
            )
            out_ref[:, :, pl.ds(s * CHUNK, CHUNK)] = jnp.transpose(
                flat, (1, 2, 0)
            )

    return pl.pallas_call(
        body,
        grid=(n_blk // group,),
        in_specs=[pl.BlockSpec((group * rows_per_blk, 128), lambda i: (i, 0))],
        out_specs=pl.BlockSpec(
            (b1, features, group * CHUNK), lambda i: (0, 0, i)
        ),
        out_shape=jax.ShapeDtypeStruct((b1, features, b0), jnp.float32),
    )(rows_packed)


def kernel(x, embedding):
    b0, b1 = x.shape
    features = embedding.shape[1]
    total = b0 * b1
    n_chunks = total // CHUNK
    per_w = n_chunks // NUM_WORKERS
    n_groups = per_w // NBUF

    table, half = _transpose_table(embedding.T)
    xi = x.astype(jnp.int32)
    idx = jnp.where(xi < half, 2 * xi, 2 * (xi - half) + 1).reshape(
        n_chunks, CHUNK
    )

    mesh = plsc.VectorSubcoreMesh(core_axis_name="c", subcore_axis_name="s")

    @functools.partial(
        pl.kernel,
        out_type=jax.ShapeDtypeStruct((n_chunks, CHUNK, features), jnp.float32),
        mesh=mesh,
        scratch_types=[
            pltpu.VMEM((per_w, CHUNK), jnp.int32),
            pltpu.VMEM((NBUF, CHUNK, features), jnp.float32),
            pltpu.SemaphoreType.DMA((NBUF,)),
            pltpu.SemaphoreType.DMA((NBUF,)),
        ],
        compiler_params=pltpu.CompilerParams(use_tc_tiling_on_sc=False),
    )
    def embed_gather(table_hbm, idx_hbm, out_hbm, idx_v, rows_v, gsem, wsem):
        w = lax.axis_index("s") * NUM_CORES + lax.axis_index("c")
        base = w * per_w
        pltpu.sync_copy(idx_hbm.at[pl.ds(base, per_w)], idx_v)

        def group(g0, carry):
            k0 = g0 * NBUF
            gathers = []
            for b in range(NBUF):
                gathers.append(
                    pltpu.async_copy(
                        table_hbm.at[idx_v.at[k0 + b]],
                        rows_v.at[b],
                        gsem.at[b],
                    )
                )
            writes = []
            for b in range(NBUF):
                gathers[b].wait()
                writes.append(
                    pltpu.async_copy(
                        rows_v.at[b],
                        out_hbm.at[base + k0 + b],
                        wsem.at[b],
                    )
                )
            for wd in writes:
                wd.wait()
            return carry

        lax.fori_loop(0, n_groups, group, 0)

    rows = embed_gather(table, idx)
    rows_packed = rows.reshape(total * features // 128, 128)
    out_t = _transpose_out(rows_packed, b0, b1, features)
    return jnp.transpose(out_t, (2, 0, 1))
